# baseline (speedup 1.0000x reference)
import jax
import jax.numpy as jnp
from jax import lax
from jax.experimental import pallas as pl
from jax.experimental.pallas import tpu as pltpu
from jax.experimental.pallas import tpu_sc as plsc

_N = 10000
_E = 320000
_E2 = 327680
_NP = _N + 16
_G = 64
_NW = 32
_CH = 128


def _mesh():
    return plsc.VectorSubcoreMesh(
        core_axis_name="c", subcore_axis_name="s", num_cores=2, num_subcores=16
    )


def _sc_gather(table, idx2):
    n_ch_tot, ch = idx2.shape
    d = table.shape[1]
    per_w = n_ch_tot // _NW
    m = n_ch_tot * ch

    def body(t_hbm, i_hbm, o_hbm, idx2d, r0, r1, r2, r3, g0, g1, g2, g3, s0, s1, s2, s3):
        rows = (r0, r1, r2, r3)
        gs = (g0, g1, g2, g3)
        ss = (s0, s1, s2, s3)
        wid = lax.axis_index("s") * 2 + lax.axis_index("c")
        cbase = wid * per_w
        ebase = cbase * ch
        pltpu.sync_copy(i_hbm.at[pl.ds(cbase, per_w)], idx2d)

        def g_start(i, b):
            pltpu.make_async_copy(t_hbm.at[idx2d.at[i]], rows[b], gs[b]).start()

        def g_wait(i, b):
            pltpu.make_async_copy(t_hbm.at[idx2d.at[i]], rows[b], gs[b]).wait()

        def s_start(i, b):
            pltpu.make_async_copy(
                rows[b], o_hbm.at[pl.ds(ebase + i * ch, ch)], ss[b]
            ).start()

        def s_wait(i, b):
            pltpu.make_async_copy(
                rows[b], o_hbm.at[pl.ds(ebase + i * ch, ch)], ss[b]
            ).wait()

        g_start(0, 0)
        g_start(1, 1)

        def step(j, c):
            for b in range(4):
                i = 4 * j + b
                g_wait(i, b)
                s_start(i, b)
                bn = (b + 2) % 4

                @pl.when(i >= 2)
                def _():
                    s_wait(i - 2, bn)

                @pl.when(i + 2 < per_w)
                def _():
                    g_start(i + 2, bn)
            return c

        lax.fori_loop(0, per_w // 4, step, 0)
        s_wait(per_w - 2, (per_w - 2) % 4)
        s_wait(per_w - 1, (per_w - 1) % 4)

    return pl.kernel(
        body,
        out_type=jax.ShapeDtypeStruct((m, d), jnp.float32),
        mesh=_mesh(),
        compiler_params=pltpu.CompilerParams(use_tc_tiling_on_sc=False, skip_device_barrier=True),
        scratch_types=[
            pltpu.VMEM((per_w, ch), jnp.int32),
        ]
        + [pltpu.VMEM((ch, d), jnp.float32)] * 4
        + [pltpu.SemaphoreType.DMA] * 8,
    )(table, idx2)


def _sc_scatter(m2, dsp2, zer):
    do = m2.shape[1]
    per_w = (m2.shape[0] // _CH) // _NW
    nseg = _NP // 16

    def body(m_hbm, d_hbm, z_hbm, o_hbm, idx2d, r0, r1, l0, l1, acc_sh):
        rows = (r0, r1)
        ls = (l0, l1)
        cid = lax.axis_index("c")
        sid = lax.axis_index("s")
        wid = sid * 2 + cid
        seg = pl.ds(sid * nseg, nseg)
        pltpu.sync_copy(z_hbm.at[seg], acc_sh.at[seg])
        cbase = wid * per_w
        ebase = cbase * _CH
        pltpu.sync_copy(d_hbm.at[pl.ds(cbase, per_w)], idx2d)
        plsc.subcore_barrier()

        def l_start(i, b):
            pltpu.make_async_copy(
                m_hbm.at[pl.ds(ebase + i * _CH, _CH)], rows[b], ls[b]
            ).start()

        def l_wait(i, b):
            pltpu.make_async_copy(
                m_hbm.at[pl.ds(ebase + i * _CH, _CH)], rows[b], ls[b]
            ).wait()

        l_start(0, 0)
        l_start(1, 1)

        def step(j, c):
            for b in range(2):
                i = 2 * j + b
                l_wait(i, b)
                pltpu.sync_copy(rows[b], acc_sh.at[idx2d.at[i]], add=True)

                @pl.when(i + 2 < per_w)
                def _():
                    l_start(i + 2, b)
            return c

        lax.fori_loop(0, per_w // 2, step, 0)
        plsc.subcore_barrier()
        pltpu.sync_copy(acc_sh.at[seg], o_hbm.at[cid, seg])

    return pl.kernel(
        body,
        out_type=jax.ShapeDtypeStruct((2, _NP, do), jnp.float32),
        mesh=_mesh(),
        compiler_params=pltpu.CompilerParams(use_tc_tiling_on_sc=False, skip_device_barrier=True),
        scratch_types=[
            pltpu.VMEM((per_w, _CH), jnp.int32),
            pltpu.VMEM((_CH, do), jnp.float32),
            pltpu.VMEM((_CH, do), jnp.float32),
            pltpu.SemaphoreType.DMA,
            pltpu.SemaphoreType.DMA,
            pltpu.VMEM_SHARED((_NP, do), jnp.float32),
        ],
    )(m2, dsp2, zer)


def _node_add(p0, p1):
    fin = p0.shape[2]

    def body(a_ref, b_ref, h_ref):
        h_ref[...] = (a_ref[0, :_N, :] + a_ref[1, :_N, :]) + (
            b_ref[0, :_N, :] + b_ref[1, :_N, :]
        )

    return pl.pallas_call(
        body, out_shape=jax.ShapeDtypeStruct((_N, fin), jnp.float32)
    )(p0, p1)


def _edge_mlp(g, wa, ba, wb, bb, be):
    f = g.shape[1]
    do = wb.shape[1]
    nblk = g.shape[0] // (2 * be)

    def body(a_ref, b_ref, wa_ref, ba_ref, wb_ref, bb_ref, o_ref):
        hi = a_ref[...]
        hj = b_ref[...]
        hb = hi.astype(jnp.bfloat16)
        db = (hj - hi).astype(jnp.bfloat16)
        wt = wa_ref[:f, :].astype(jnp.bfloat16)
        wu = wa_ref[f:, :].astype(jnp.bfloat16)
        pre = (
            jnp.dot(hb, wt, preferred_element_type=jnp.float32)
            + jnp.dot(db, wu, preferred_element_type=jnp.float32)
            + ba_ref[...]
        )
        m1 = jnp.maximum(pre, 0.0)
        o_ref[...] = jnp.maximum(
            jnp.dot(
                m1.astype(jnp.bfloat16),
                wb_ref[...].astype(jnp.bfloat16),
                preferred_element_type=jnp.float32,
            )
            + bb_ref[...],
            0.0,
        )

    return pl.pallas_call(
        body,
        grid=(nblk,),
        in_specs=[
            pl.BlockSpec((be, f), lambda i: (i, 0)),
            pl.BlockSpec((be, f), lambda i, _n=nblk: (i + _n, 0)),
            pl.BlockSpec((2 * f, 64), lambda i: (0, 0)),
            pl.BlockSpec((1, 64), lambda i: (0, 0)),
            pl.BlockSpec((64, do), lambda i: (0, 0)),
            pl.BlockSpec((1, do), lambda i: (0, 0)),
        ],
        out_specs=pl.BlockSpec((be, do), lambda i: (i, 0)),
        out_shape=jax.ShapeDtypeStruct((g.shape[0] // 2, do), jnp.float32),
    )(g, g, wa, ba.reshape(1, 64), wb, bb.reshape(1, do))


def _pool_mlp(p0, p1, b2d, wl1, bl1, wl2, bl2):
    def body(p_ref, q_ref, b_ref, w1_ref, b1_ref, w2_ref, b2_ref, o_ref):
        h = (p_ref[0, :_N, :] + p_ref[1, :_N, :]) + (
            q_ref[0, :_N, :] + q_ref[1, :_N, :]
        )
        gid = lax.broadcasted_iota(jnp.int32, (_G, _N), 0)
        oh = (gid == b_ref[...]).astype(jnp.float32)
        sums = jnp.dot(
            oh, h, preferred_element_type=jnp.float32,
            precision=jax.lax.Precision.HIGHEST,
        )
        counts = jnp.sum(oh, axis=1, keepdims=True)
        pooled = sums / jnp.maximum(counts, 1.0)
        z = jnp.maximum(
            jnp.dot(
                pooled.astype(jnp.bfloat16),
                w1_ref[...].astype(jnp.bfloat16),
                preferred_element_type=jnp.float32,
            )
            + b1_ref[...],
            0.0,
        )
        z = jnp.maximum(
            jnp.dot(
                z.astype(jnp.bfloat16),
                w2_ref[...].astype(jnp.bfloat16),
                preferred_element_type=jnp.float32,
            )
            + b2_ref[...],
            0.0,
        )
        mx = jnp.max(z, axis=1, keepdims=True)
        lse = mx + jnp.log(jnp.sum(jnp.exp(z - mx), axis=1, keepdims=True))
        o_ref[...] = z - lse

    return pl.pallas_call(
        body, out_shape=jax.ShapeDtypeStruct((_G, 2), jnp.float32)
    )(p0, p1, b2d, wl1, bl1.reshape(1, 32), wl2, bl2.reshape(1, 2))


def kernel(x, edge_index, batch, W1_1, b1_1, W1_2, b1_2, W2_1, b2_1, W2_2, b2_2, W3_1, b3_1, W3_2, b3_2, W4_1, b4_1, W4_2, b4_2, W5_1, b5_1, W5_2, b5_2, Wl1, bl1, Wl2, bl2):
    src = edge_index[0]
    dst = edge_index[1]
    eh = _E2 // 2
    padz = jnp.zeros((_E2 - _E,), jnp.int32)
    jg1 = jnp.concatenate([dst[:eh], src[:eh]]).reshape(2 * eh // _CH, _CH)
    jg2 = jnp.concatenate([dst[eh:], padz, src[eh:], padz]).reshape(
        2 * eh // _CH, _CH
    )
    dsp1 = dst[:eh].reshape(eh // _CH, _CH)
    dsp2 = jnp.concatenate([dst[eh:], jnp.full((_E2 - _E,), _N, jnp.int32)]).reshape(
        eh // _CH, _CH
    )
    z32 = jnp.zeros((_NP, 32), jnp.float32)
    z64 = jnp.zeros((_NP, 64), jnp.float32)

    tbl = x
    pp = None
    for li, (wa, ba, wb, bb, zz, be) in enumerate((
        (W1_1, b1_1, W1_2, b1_2, z32, 4096),
        (W2_1, b2_1, W2_2, b2_2, z32, 8192),
        (W3_1, b3_1, W3_2, b3_2, z32, 8192),
        (W4_1, b4_1, W4_2, b4_2, z32, 8192),
        (W5_1, b5_1, W5_2, b5_2, z64, 8192),
    )):
        pp = []
        for jg, dsp in ((jg1, dsp1), (jg2, dsp2)):
            g = _sc_gather(tbl, jg)
            m2 = _edge_mlp(g, wa, ba, wb, bb, be)
            pp.append(_sc_scatter(m2, dsp, zz))
        if li < 4:
            tbl = _node_add(pp[0], pp[1])

    return _pool_mlp(pp[0], pp[1], batch.reshape(1, _N), Wl1, bl1, Wl2, bl2)


# speedup vs baseline: 1.0079x; 1.0079x over previous
import jax
import jax.numpy as jnp
from jax import lax
from jax.experimental import pallas as pl
from jax.experimental.pallas import tpu as pltpu
from jax.experimental.pallas import tpu_sc as plsc

_N = 10000
_E = 320000
_E2 = 327680
_NP = _N + 16
_G = 64
_NW = 32
_CH = 128


def _mesh():
    return plsc.VectorSubcoreMesh(
        core_axis_name="c", subcore_axis_name="s", num_cores=2, num_subcores=16
    )


def _sc_gather(table, idx2):
    n_ch_tot, ch = idx2.shape
    d = table.shape[1]
    per_w = n_ch_tot // _NW
    m = n_ch_tot * ch

    def body(t_hbm, i_hbm, o_hbm, idx2d, r0, r1, r2, r3, g0, g1, g2, g3, s0, s1, s2, s3):
        rows = (r0, r1, r2, r3)
        gs = (g0, g1, g2, g3)
        ss = (s0, s1, s2, s3)
        wid = lax.axis_index("s") * 2 + lax.axis_index("c")
        cbase = wid * per_w
        ebase = cbase * ch
        pltpu.sync_copy(i_hbm.at[pl.ds(cbase, per_w)], idx2d)

        def g_start(i, b):
            pltpu.make_async_copy(t_hbm.at[idx2d.at[i]], rows[b], gs[b]).start()

        def g_wait(i, b):
            pltpu.make_async_copy(t_hbm.at[idx2d.at[i]], rows[b], gs[b]).wait()

        def s_start(i, b):
            pltpu.make_async_copy(
                rows[b], o_hbm.at[pl.ds(ebase + i * ch, ch)], ss[b]
            ).start()

        def s_wait(i, b):
            pltpu.make_async_copy(
                rows[b], o_hbm.at[pl.ds(ebase + i * ch, ch)], ss[b]
            ).wait()

        g_start(0, 0)
        g_start(1, 1)

        def step(j, c):
            for b in range(4):
                i = 4 * j + b
                g_wait(i, b)
                s_start(i, b)
                bn = (b + 2) % 4

                @pl.when(i >= 2)
                def _():
                    s_wait(i - 2, bn)

                @pl.when(i + 2 < per_w)
                def _():
                    g_start(i + 2, bn)
            return c

        lax.fori_loop(0, per_w // 4, step, 0)
        s_wait(per_w - 2, (per_w - 2) % 4)
        s_wait(per_w - 1, (per_w - 1) % 4)

    return pl.kernel(
        body,
        out_type=jax.ShapeDtypeStruct((m, d), jnp.float32),
        mesh=_mesh(),
        compiler_params=pltpu.CompilerParams(use_tc_tiling_on_sc=False, skip_device_barrier=True),
        scratch_types=[
            pltpu.VMEM((per_w, ch), jnp.int32),
        ]
        + [pltpu.VMEM((ch, d), jnp.float32)] * 4
        + [pltpu.SemaphoreType.DMA] * 8,
    )(table, idx2)


def _sc_scatter(m2, dsp2, zer):
    do = m2.shape[1]
    per_w = (m2.shape[0] // _CH) // _NW
    nseg = _NP // 16

    def body(m_hbm, d_hbm, z_hbm, o_hbm, idx2d, r0, r1, l0, l1, acc_sh):
        rows = (r0, r1)
        ls = (l0, l1)
        cid = lax.axis_index("c")
        sid = lax.axis_index("s")
        wid = sid * 2 + cid
        seg = pl.ds(sid * nseg, nseg)
        pltpu.sync_copy(z_hbm.at[cid, seg], acc_sh.at[seg])
        cbase = wid * per_w
        ebase = cbase * _CH
        pltpu.sync_copy(d_hbm.at[pl.ds(cbase, per_w)], idx2d)
        plsc.subcore_barrier()

        def l_start(i, b):
            pltpu.make_async_copy(
                m_hbm.at[pl.ds(ebase + i * _CH, _CH)], rows[b], ls[b]
            ).start()

        def l_wait(i, b):
            pltpu.make_async_copy(
                m_hbm.at[pl.ds(ebase + i * _CH, _CH)], rows[b], ls[b]
            ).wait()

        l_start(0, 0)
        l_start(1, 1)

        def step(j, c):
            for b in range(2):
                i = 2 * j + b
                l_wait(i, b)
                pltpu.sync_copy(rows[b], acc_sh.at[idx2d.at[i]], add=True)

                @pl.when(i + 2 < per_w)
                def _():
                    l_start(i + 2, b)
            return c

        lax.fori_loop(0, per_w // 2, step, 0)
        plsc.subcore_barrier()
        pltpu.sync_copy(acc_sh.at[seg], o_hbm.at[cid, seg])

    return pl.kernel(
        body,
        out_type=jax.ShapeDtypeStruct((2, _NP, do), jnp.float32),
        mesh=_mesh(),
        compiler_params=pltpu.CompilerParams(use_tc_tiling_on_sc=False, skip_device_barrier=True),
        scratch_types=[
            pltpu.VMEM((per_w, _CH), jnp.int32),
            pltpu.VMEM((_CH, do), jnp.float32),
            pltpu.VMEM((_CH, do), jnp.float32),
            pltpu.SemaphoreType.DMA,
            pltpu.SemaphoreType.DMA,
            pltpu.VMEM_SHARED((_NP, do), jnp.float32),
        ],
    )(m2, dsp2, zer)


def _node_add(p):
    fin = p.shape[2]

    def body(a_ref, h_ref):
        h_ref[...] = a_ref[0, :_N, :] + a_ref[1, :_N, :]

    return pl.pallas_call(
        body, out_shape=jax.ShapeDtypeStruct((_N, fin), jnp.float32)
    )(p)


def _edge_mlp(g, wa, ba, wb, bb, be):
    f = g.shape[1]
    do = wb.shape[1]
    nblk = g.shape[0] // (2 * be)

    def body(a_ref, b_ref, wa_ref, ba_ref, wb_ref, bb_ref, o_ref):
        hi = a_ref[...]
        hj = b_ref[...]
        mc = jnp.concatenate(
            [hi.astype(jnp.bfloat16), (hj - hi).astype(jnp.bfloat16)], axis=1
        )
        pre = (
            jnp.dot(
                mc,
                wa_ref[...].astype(jnp.bfloat16),
                preferred_element_type=jnp.float32,
            )
            + ba_ref[...]
        )
        m1 = jnp.maximum(pre, 0.0)
        o_ref[...] = jnp.maximum(
            jnp.dot(
                m1.astype(jnp.bfloat16),
                wb_ref[...].astype(jnp.bfloat16),
                preferred_element_type=jnp.float32,
            )
            + bb_ref[...],
            0.0,
        )

    return pl.pallas_call(
        body,
        grid=(nblk,),
        in_specs=[
            pl.BlockSpec((be, f), lambda i: (i, 0)),
            pl.BlockSpec((be, f), lambda i, _n=nblk: (i + _n, 0)),
            pl.BlockSpec((2 * f, 64), lambda i: (0, 0)),
            pl.BlockSpec((1, 64), lambda i: (0, 0)),
            pl.BlockSpec((64, do), lambda i: (0, 0)),
            pl.BlockSpec((1, do), lambda i: (0, 0)),
        ],
        out_specs=pl.BlockSpec((be, do), lambda i: (i, 0)),
        out_shape=jax.ShapeDtypeStruct((g.shape[0] // 2, do), jnp.float32),
    )(g, g, wa, ba.reshape(1, 64), wb, bb.reshape(1, do))


def _pool_mlp(p, b2d, wl1, bl1, wl2, bl2):
    def body(p_ref, b_ref, w1_ref, b1_ref, w2_ref, b2_ref, o_ref):
        h = p_ref[0, :_N, :] + p_ref[1, :_N, :]
        gid = lax.broadcasted_iota(jnp.int32, (_G, _N), 0)
        oh = (gid == b_ref[...]).astype(jnp.float32)
        sums = jnp.dot(
            oh, h, preferred_element_type=jnp.float32,
            precision=jax.lax.Precision.HIGHEST,
        )
        counts = jnp.sum(oh, axis=1, keepdims=True)
        pooled = sums / jnp.maximum(counts, 1.0)
        z = jnp.maximum(
            jnp.dot(
                pooled.astype(jnp.bfloat16),
                w1_ref[...].astype(jnp.bfloat16),
                preferred_element_type=jnp.float32,
            )
            + b1_ref[...],
            0.0,
        )
        z = jnp.maximum(
            jnp.dot(
                z.astype(jnp.bfloat16),
                w2_ref[...].astype(jnp.bfloat16),
                preferred_element_type=jnp.float32,
            )
            + b2_ref[...],
            0.0,
        )
        mx = jnp.max(z, axis=1, keepdims=True)
        lse = mx + jnp.log(jnp.sum(jnp.exp(z - mx), axis=1, keepdims=True))
        o_ref[...] = z - lse

    return pl.pallas_call(
        body, out_shape=jax.ShapeDtypeStruct((_G, 2), jnp.float32)
    )(p, b2d, wl1, bl1.reshape(1, 32), wl2, bl2.reshape(1, 2))


def kernel(x, edge_index, batch, W1_1, b1_1, W1_2, b1_2, W2_1, b2_1, W2_2, b2_2, W3_1, b3_1, W3_2, b3_2, W4_1, b4_1, W4_2, b4_2, W5_1, b5_1, W5_2, b5_2, Wl1, bl1, Wl2, bl2):
    src = edge_index[0]
    dst = edge_index[1]
    eh = _E2 // 2
    padz = jnp.zeros((_E2 - _E,), jnp.int32)
    jg1 = jnp.concatenate([dst[:eh], src[:eh]]).reshape(2 * eh // _CH, _CH)
    jg2 = jnp.concatenate([dst[eh:], padz, src[eh:], padz]).reshape(
        2 * eh // _CH, _CH
    )
    dsp1 = dst[:eh].reshape(eh // _CH, _CH)
    dsp2 = jnp.concatenate([dst[eh:], jnp.full((_E2 - _E,), _N, jnp.int32)]).reshape(
        eh // _CH, _CH
    )
    z32 = jnp.zeros((2, _NP, 32), jnp.float32)
    z64 = jnp.zeros((2, _NP, 64), jnp.float32)

    tbl = x
    p = None
    for li, (wa, ba, wb, bb, zz, be) in enumerate((
        (W1_1, b1_1, W1_2, b1_2, z32, 4096),
        (W2_1, b2_1, W2_2, b2_2, z32, 8192),
        (W3_1, b3_1, W3_2, b3_2, z32, 8192),
        (W4_1, b4_1, W4_2, b4_2, z32, 8192),
        (W5_1, b5_1, W5_2, b5_2, z64, 8192),
    )):
        p = zz
        for jg, dsp in ((jg1, dsp1), (jg2, dsp2)):
            g = _sc_gather(tbl, jg)
            m2 = _edge_mlp(g, wa, ba, wb, bb, be)
            p = _sc_scatter(m2, dsp, p)
        if li < 4:
            tbl = _node_add(p)

    return _pool_mlp(p, batch.reshape(1, _N), Wl1, bl1, Wl2, bl2)
